# baseline (speedup 1.0000x reference)
import functools
import math

import jax
import jax.numpy as jnp
from jax import lax
from jax.experimental import pallas as pl
from jax.experimental.pallas import tpu as pltpu
from jax.experimental.pallas import tpu_sc as plsc

EMB = 64
LANES = 16
CHUNK = 128


def _make_sc_gather(num_workers: int, nchunk: int, scale: float):
    mesh = plsc.VectorSubcoreMesh(core_axis_name="c", subcore_axis_name="s")
    b_per_w = nchunk * CHUNK

    @functools.partial(
        pl.kernel,
        mesh=mesh,
        out_type=jax.ShapeDtypeStruct((num_workers * b_per_w, EMB), jnp.float32),
        scratch_types=[
            pltpu.VMEM((nchunk, CHUNK), jnp.int32),
            pltpu.VMEM((CHUNK, EMB), jnp.float32),
            pltpu.SemaphoreType.DMA,
        ],
        compiler_params=pltpu.CompilerParams(use_tc_tiling_on_sc=False),
    )
    def sc_embed(tokens_hbm, table_hbm, out_hbm, idx_v, rows_v, sem):
        nc = lax.axis_size("c")
        wid = lax.axis_index("s") * nc + lax.axis_index("c")
        pltpu.sync_copy(tokens_hbm.at[wid], idx_v)

        def step(j, carry):
            pltpu.async_copy(table_hbm.at[idx_v.at[j]], rows_v, sem).wait()

            def scale_row(r, c2):
                for t in range(EMB // LANES):
                    sl = pl.ds(t * LANES, LANES)
                    rows_v[r, sl] = rows_v[r, sl] * scale
                return c2

            lax.fori_loop(0, CHUNK, scale_row, 0, unroll=4)
            base = wid * b_per_w + j * CHUNK
            pltpu.sync_copy(rows_v, out_hbm.at[pl.ds(base, CHUNK)])
            return carry

        lax.fori_loop(0, nchunk, step, 0)

    return sc_embed


def kernel(tokens, table):
    bsz, seq = tokens.shape
    total = bsz * seq
    num_workers = 32
    assert total % (num_workers * CHUNK) == 0
    nchunk = total // (num_workers * CHUNK)
    scale = math.sqrt(float(EMB))
    toks = tokens.reshape(num_workers, nchunk, CHUNK).astype(jnp.int32)
    out = _make_sc_gather(num_workers, nchunk, scale)(toks, table)
    return out.reshape(bsz, seq, EMB)
